# baseline (speedup 1.0000x reference)
import functools

import jax
import jax.numpy as jnp
from jax import lax
from jax.experimental import pallas as pl
from jax.experimental.pallas import tpu as pltpu
from jax.experimental.pallas import tpu_sc as plsc

_NUM_CH_NORM = 64.0


def _sc_gather_bias(channel_ids, region_ids, channel_emb, region_emb):
    ch = channel_ids.shape[0]
    d = channel_emb.shape[1]
    n_workers = 4
    per = ch // n_workers
    nc = 1

    mesh = plsc.VectorSubcoreMesh(core_axis_name="c", subcore_axis_name="s", num_cores=nc)

    @functools.partial(
        pl.kernel,
        out_type=jax.ShapeDtypeStruct((ch, d), jnp.float32),
        mesh=mesh,
        scratch_types=[
            pltpu.VMEM((per,), jnp.int32),
            pltpu.VMEM((per,), jnp.int32),
            pltpu.VMEM((per, d), jnp.float32),
            pltpu.VMEM((per, d), jnp.float32),
            pltpu.SemaphoreType.DMA,
            pltpu.SemaphoreType.DMA,
        ],
    )
    def gather_kernel(cid_hbm, rid_hbm, chemb_hbm, rgemb_hbm, out_hbm,
                      cidx_v, ridx_v, chrows_v, rgrows_v, sem_a, sem_b):
        wid = lax.axis_index("s") * nc + lax.axis_index("c")

        @pl.when(wid < n_workers)
        def _():
            base = wid * per
            pltpu.sync_copy(cid_hbm.at[pl.ds(base, per)], cidx_v)
            pltpu.sync_copy(rid_hbm.at[pl.ds(base, per)], ridx_v)
            cp_a = pltpu.async_copy(chemb_hbm.at[cidx_v], chrows_v, sem_a)
            cp_b = pltpu.async_copy(rgemb_hbm.at[ridx_v], rgrows_v, sem_b)
            cp_a.wait()
            cp_b.wait()
            for r in range(per):
                for j in range(d // 16):
                    sl = pl.ds(j * 16, 16)
                    chrows_v[r, sl] = chrows_v[r, sl] + rgrows_v[r, sl]
            pltpu.sync_copy(chrows_v, out_hbm.at[pl.ds(base, per)])

    return gather_kernel(channel_ids, region_ids, channel_emb, region_emb)


def _tc_add(xr, scb, coords, coord_w, coord_b, mm, mask_w, mask_b,
            count_w, count_b, cc, ch_blk, t_blk):
    n_rows, t, d = xr.shape
    ch = scb.shape[0]
    nbc = ch // ch_blk
    grid = (n_rows // ch_blk, t // t_blk)

    def add_body(scb_ref, coords_ref, cw_ref, cb_ref, mm_ref, mw_ref, mb_ref,
                 ctw_ref, ctb_ref, x_ref, o_ref):
        coord_bias = jnp.dot(coords_ref[...], cw_ref[...],
                             preferred_element_type=jnp.float32)
        mask_bias = jnp.dot(mm_ref[...], mw_ref[...],
                            preferred_element_type=jnp.float32)
        bias = (scb_ref[...] + coord_bias + cb_ref[...] + mask_bias
                + mb_ref[...] + cc * ctw_ref[...] + ctb_ref[...])
        o_ref[...] = x_ref[...] + bias[:, None, :]

    row_spec = pl.BlockSpec((ch_blk, t_blk, d), lambda i, j: (i, j, 0))
    bias_spec = pl.BlockSpec((ch_blk, d), lambda i, j: (i % nbc, 0))
    coords_spec = pl.BlockSpec((ch_blk, coords.shape[1]),
                               lambda i, j: (i % nbc, 0))
    mm_spec = pl.BlockSpec((ch_blk, 1), lambda i, j: (i % nbc, 0))

    def full(a):
        return pl.BlockSpec(a.shape, lambda i, j: (0,) * a.ndim)

    return pl.pallas_call(
        add_body,
        grid=grid,
        in_specs=[bias_spec, coords_spec, full(coord_w), full(coord_b),
                  mm_spec, full(mask_w), full(mask_b), full(count_w),
                  full(count_b), row_spec],
        out_specs=row_spec,
        out_shape=jax.ShapeDtypeStruct((n_rows, t, d), jnp.float32),
    )(scb, coords, coord_w, coord_b, mm, mask_w, mask_b, count_w, count_b, xr)


def kernel(x, channel_emb, region_emb, coord_w, coord_b, mask_w, mask_b,
           count_w, count_b, coords, montage_mask, channel_ids, region_ids):
    b, ch, t, d = x.shape
    cc = float(ch) / max(_NUM_CH_NORM, 1.0)

    scb = _sc_gather_bias(channel_ids.astype(jnp.int32),
                          region_ids.astype(jnp.int32),
                          channel_emb, region_emb)

    xr = x.reshape(b * ch, t, d)
    out = _tc_add(xr, scb, coords, coord_w, coord_b.reshape(1, d),
                  montage_mask.reshape(ch, 1), mask_w, mask_b.reshape(1, d),
                  count_w, count_b.reshape(1, d), cc, ch_blk=16, t_blk=256)
    return out.reshape(b, ch, t, d)


# speedup vs baseline: 1.1916x; 1.1916x over previous
import functools

import jax
import jax.numpy as jnp
from jax import lax
from jax.experimental import pallas as pl
from jax.experimental.pallas import tpu as pltpu
from jax.experimental.pallas import tpu_sc as plsc

_NUM_CH_NORM = 64.0


def _sc_gather_bias(channel_ids, region_ids, channel_emb, region_emb):
    ch = channel_ids.shape[0]
    d = channel_emb.shape[1]
    n_workers = 4
    per = ch // n_workers
    nc = 1

    mesh = plsc.VectorSubcoreMesh(core_axis_name="c", subcore_axis_name="s", num_cores=nc)

    @functools.partial(
        pl.kernel,
        out_type=jax.ShapeDtypeStruct((ch, d), jnp.float32),
        mesh=mesh,
        scratch_types=[
            pltpu.VMEM((per,), jnp.int32),
            pltpu.VMEM((per,), jnp.int32),
            pltpu.VMEM((per, d), jnp.float32),
            pltpu.VMEM((per, d), jnp.float32),
            pltpu.SemaphoreType.DMA,
            pltpu.SemaphoreType.DMA,
        ],
    )
    def gather_kernel(cid_hbm, rid_hbm, chemb_hbm, rgemb_hbm, out_hbm,
                      cidx_v, ridx_v, chrows_v, rgrows_v, sem_a, sem_b):
        wid = lax.axis_index("s") * nc + lax.axis_index("c")

        @pl.when(wid < n_workers)
        def _():
            base = wid * per
            pltpu.sync_copy(cid_hbm.at[pl.ds(base, per)], cidx_v)
            pltpu.sync_copy(rid_hbm.at[pl.ds(base, per)], ridx_v)
            cp_a = pltpu.async_copy(chemb_hbm.at[cidx_v], chrows_v, sem_a)
            cp_b = pltpu.async_copy(rgemb_hbm.at[ridx_v], rgrows_v, sem_b)
            cp_a.wait()
            cp_b.wait()
            for r in range(per):
                for j in range(d // 16):
                    sl = pl.ds(j * 16, 16)
                    chrows_v[r, sl] = chrows_v[r, sl] + rgrows_v[r, sl]
            pltpu.sync_copy(chrows_v, out_hbm.at[pl.ds(base, per)])

    return gather_kernel(channel_ids, region_ids, channel_emb, region_emb)


def _tc_add(xr, scb, coords, coord_w, coord_b, mm, mask_w, mask_b,
            count_w, count_b, cc, ch_blk, t_blk):
    n_rows, t, d = xr.shape
    ch = scb.shape[0]
    nbc = ch // ch_blk
    grid = (n_rows // ch_blk, t // t_blk)

    def add_body(scb_ref, coords_ref, cw_ref, cb_ref, mm_ref, mw_ref, mb_ref,
                 ctw_ref, ctb_ref, x_ref, o_ref):
        coord_bias = jnp.dot(coords_ref[...], cw_ref[...],
                             preferred_element_type=jnp.float32)
        mask_bias = jnp.dot(mm_ref[...], mw_ref[...],
                            preferred_element_type=jnp.float32)
        bias = (scb_ref[...] + coord_bias + cb_ref[...] + mask_bias
                + mb_ref[...] + cc * ctw_ref[...] + ctb_ref[...])
        o_ref[...] = x_ref[...] + bias[:, None, :]

    row_spec = pl.BlockSpec((ch_blk, t_blk, d), lambda i, j: (i, j, 0))
    bias_spec = pl.BlockSpec((ch_blk, d), lambda i, j: (i % nbc, 0))
    coords_spec = pl.BlockSpec((ch_blk, coords.shape[1]),
                               lambda i, j: (i % nbc, 0))
    mm_spec = pl.BlockSpec((ch_blk, 1), lambda i, j: (i % nbc, 0))

    def full(a):
        return pl.BlockSpec(a.shape, lambda i, j: (0,) * a.ndim)

    return pl.pallas_call(
        add_body,
        grid=grid,
        in_specs=[bias_spec, coords_spec, full(coord_w), full(coord_b),
                  mm_spec, full(mask_w), full(mask_b), full(count_w),
                  full(count_b), row_spec],
        out_specs=row_spec,
        out_shape=jax.ShapeDtypeStruct((n_rows, t, d), jnp.float32),
    )(scb, coords, coord_w, coord_b, mm, mask_w, mask_b, count_w, count_b, xr)


def kernel(x, channel_emb, region_emb, coord_w, coord_b, mask_w, mask_b,
           count_w, count_b, coords, montage_mask, channel_ids, region_ids):
    b, ch, t, d = x.shape
    cc = float(ch) / max(_NUM_CH_NORM, 1.0)

    scb = (jnp.take(channel_emb, channel_ids, axis=0)
           + jnp.take(region_emb, region_ids, axis=0))

    xr = x.reshape(b * ch, t, d)
    out = _tc_add(xr, scb, coords, coord_w, coord_b.reshape(1, d),
                  montage_mask.reshape(ch, 1), mask_w, mask_b.reshape(1, d),
                  count_w, count_b.reshape(1, d), cc, ch_blk=16, t_blk=256)
    return out.reshape(b, ch, t, d)
